# Initial kernel scaffold; baseline (speedup 1.0000x reference)
import jax
import jax.numpy as jnp
from jax.experimental import pallas as pl


def kernel(track_sequences, edge_index, params):
    raise NotImplementedError("write your pallas kernel here")


# speedup vs baseline: 4.7883x; 4.7883x over previous
import functools

import jax
import jax.numpy as jnp
from jax import lax
from jax.experimental import pallas as pl
from jax.experimental.pallas import tpu as pltpu
from jax.experimental.pallas import tpu_sc as plsc

_N = 10000
_T = 20
_IN = 16
_HID = 128
_LAT = 64
_E = 320000

_NB = 400
_CH = 128
_ECH = 2560
_EPAD = _ECH * _CH
_EB = 2048
_CPT = _ECH // 32
_NPAD = 10240
_RPT = _NPAD // 16
_ZCH = 128
_NBUF = 4

_f32 = jnp.float32


def _encoder_body(x_ref, wih0, whh0, b0, wih1, whh1, b1, we1, be1, we2, be2,
                  lat_ref):
    x = x_ref[...]
    nb = x.shape[0]
    h = jnp.zeros((nb, _HID), _f32)
    c = jnp.zeros((nb, _HID), _f32)
    hs = []
    for t in range(_T):
        xt = x[:, t * _IN:(t + 1) * _IN]
        g = (jnp.dot(xt, wih0[...], preferred_element_type=_f32)
             + jnp.dot(h, whh0[...], preferred_element_type=_f32) + b0[...])
        gi = jax.nn.sigmoid(g[:, 0 * _HID:1 * _HID])
        gf = jax.nn.sigmoid(g[:, 1 * _HID:2 * _HID])
        gg = jnp.tanh(g[:, 2 * _HID:3 * _HID])
        go = jax.nn.sigmoid(g[:, 3 * _HID:4 * _HID])
        c = gf * c + gi * gg
        h = go * jnp.tanh(c)
        hs.append(h)
    h = jnp.zeros((nb, _HID), _f32)
    c = jnp.zeros((nb, _HID), _f32)
    for t in range(_T):
        g = (jnp.dot(hs[t], wih1[...], preferred_element_type=_f32)
             + jnp.dot(h, whh1[...], preferred_element_type=_f32) + b1[...])
        gi = jax.nn.sigmoid(g[:, 0 * _HID:1 * _HID])
        gf = jax.nn.sigmoid(g[:, 1 * _HID:2 * _HID])
        gg = jnp.tanh(g[:, 2 * _HID:3 * _HID])
        go = jax.nn.sigmoid(g[:, 3 * _HID:4 * _HID])
        c = gf * c + gi * gg
        h = go * jnp.tanh(c)
    e = jnp.maximum(jnp.dot(h, we1[...], preferred_element_type=_f32)
                    + be1[...], 0.0)
    lat_ref[...] = jnp.dot(e, we2[...], preferred_element_type=_f32) + be2[...]


def _encoder(x2, wih0, whh0, b0, wih1, whh1, b1, we1, be1, we2, be2):
    full = lambda shape: pl.BlockSpec(shape, lambda i: tuple(0 for _ in shape))
    return pl.pallas_call(
        _encoder_body,
        grid=(_N // _NB,),
        in_specs=[
            pl.BlockSpec((_NB, _T * _IN), lambda i: (i, 0)),
            full((_IN, 4 * _HID)), full((_HID, 4 * _HID)), full((1, 4 * _HID)),
            full((_HID, 4 * _HID)), full((_HID, 4 * _HID)), full((1, 4 * _HID)),
            full((_HID, _HID)), full((1, _HID)),
            full((_HID, _LAT)), full((1, _LAT)),
        ],
        out_specs=pl.BlockSpec((_NB, _LAT), lambda i: (i, 0)),
        out_shape=jax.ShapeDtypeStruct((_N, _LAT), _f32),
    )(x2, wih0, whh0, b0, wih1, whh1, b1, we1, be1, we2, be2)


@functools.cache
def _make_deg_kernel():
  mesh = plsc.VectorSubcoreMesh(core_axis_name="c", subcore_axis_name="s")

  @functools.partial(
      pl.kernel,
      out_type=jax.ShapeDtypeStruct((2, _NPAD, 16), _f32),
      mesh=mesh,
      scratch_types=[
          pltpu.VMEM((_CH, 16), _f32),
          pltpu.VMEM((_ZCH, 16), _f32),
          pltpu.VMEM((_CPT, _CH), jnp.int32),
          pltpu.VMEM_SHARED((_NPAD, 16), _f32),
      ],
  )
  def _deg_kernel(dst2_hbm, deg_hbm, ones_v, stage_v, didx_v, acc_sh):
      c = lax.axis_index("c")
      s = lax.axis_index("s")

      def fill(i, carry):
          ones_v[i, :] = jnp.full((16,), 1.0, _f32)
          return carry
      lax.fori_loop(0, _CH, fill, 0)

      def zfill(i, carry):
          stage_v[i, :] = jnp.zeros((16,), _f32)
          return carry
      lax.fori_loop(0, _ZCH, zfill, 0)

      pltpu.sync_copy(dst2_hbm.at[pl.ds(c * (_ECH // 2) + s * _CPT, _CPT)],
                      didx_v)

      row0 = s * _RPT
      for k in range(_RPT // _ZCH):
          pltpu.sync_copy(stage_v, acc_sh.at[pl.ds(row0 + k * _ZCH, _ZCH)])
      plsc.subcore_barrier()

      def chunk(g, carry):
          pltpu.sync_copy(ones_v, acc_sh.at[didx_v.at[g]], add=True)
          return carry
      lax.fori_loop(0, _CPT, chunk, 0)
      plsc.subcore_barrier()

      for k in range(_RPT // _ZCH):
          r0 = row0 + k * _ZCH
          pltpu.sync_copy(acc_sh.at[pl.ds(r0, _ZCH)], stage_v)
          pltpu.sync_copy(stage_v, deg_hbm.at[c, pl.ds(r0, _ZCH)])

  return _deg_kernel


@functools.cache
def _make_agg_kernel():
  mesh = plsc.VectorSubcoreMesh(core_axis_name="c", subcore_axis_name="s")

  @functools.partial(
      pl.kernel,
      out_type=jax.ShapeDtypeStruct((2, _NPAD, _HID), _f32),
      mesh=mesh,
      scratch_types=[
          pltpu.VMEM((4, _CH), jnp.int32),
          pltpu.VMEM((4, _CH), jnp.int32),
          pltpu.VMEM((2, _CH, _HID), _f32),
          pltpu.VMEM_SHARED((_NPAD, _HID), _f32),
      ] + [pltpu.SemaphoreType.DMA] * 6,
  )
  def _agg_kernel(y_hbm, src_hbm, dst_hbm, agg_hbm, sidx_v, didx_v, rows_v,
                  acc_sh, *sems):
      c = lax.axis_index("c")
      s = lax.axis_index("s")
      isem = sems[:4]
      gsem = sems[4:]

      def zfill(i, carry):
          for j in range(_HID // 16):
              rows_v[0, i, pl.ds(j * 16, 16)] = jnp.zeros((16,), _f32)
          return carry
      lax.fori_loop(0, _CH, zfill, 0)

      row0 = s * _RPT
      for k in range(_RPT // _ZCH):
          pltpu.sync_copy(rows_v.at[0], acc_sh.at[pl.ds(row0 + k * _ZCH,
                                                        _ZCH)])
      plsc.subcore_barrier()

      ebase = (c * (_ECH // 2) + s * _CPT) * _CH

      def start_idx(g, b):
          pltpu.async_copy(src_hbm.at[pl.ds(ebase + g * _CH, _CH)],
                           sidx_v.at[b], isem[b])
          pltpu.async_copy(dst_hbm.at[pl.ds(ebase + g * _CH, _CH)],
                           didx_v.at[b], isem[b])

      def wait_idx(g, b):
          pltpu.make_async_copy(src_hbm.at[pl.ds(ebase + g * _CH, _CH)],
                                sidx_v.at[b], isem[b]).wait()
          pltpu.make_async_copy(dst_hbm.at[pl.ds(ebase + g * _CH, _CH)],
                                didx_v.at[b], isem[b]).wait()

      def start_gather(g, bi, br):
          pltpu.async_copy(y_hbm.at[sidx_v.at[bi]], rows_v.at[br], gsem[br])

      def wait_gather(bi, br):
          pltpu.make_async_copy(y_hbm.at[sidx_v.at[bi]], rows_v.at[br],
                                gsem[br]).wait()

      for b in range(4):
          start_idx(b, b)
      for g in range(2):
          wait_idx(g, g)
          start_gather(g, g, g)

      def outer(j, carry):
          for b in range(4):
              g = j * 4 + b
              br = b % 2
              wait_gather(b, br)
              pltpu.sync_copy(rows_v.at[br], acc_sh.at[didx_v.at[b]],
                              add=True)

              @pl.when(g + 4 < _CPT)
              def _():
                  start_idx(g + 4, b)

              @pl.when(g + 2 < _CPT)
              def _():
                  wait_idx(g + 2, (b + 2) % 4)
                  start_gather(g + 2, (b + 2) % 4, br)
          return carry
      lax.fori_loop(0, _CPT // 4, outer, 0)
      plsc.subcore_barrier()

      for k in range(_RPT // _ZCH):
          r0 = row0 + k * _ZCH
          pltpu.sync_copy(acc_sh.at[pl.ds(r0, _ZCH)], rows_v.at[0])
          pltpu.sync_copy(rows_v.at[0], agg_hbm.at[c, pl.ds(r0, _ZCH)])

  return _agg_kernel


@functools.cache
def _make_edge_pre_kernel():
  mesh = plsc.VectorSubcoreMesh(core_axis_name="c", subcore_axis_name="s")

  @functools.partial(
      pl.kernel,
      out_type=jax.ShapeDtypeStruct((_EPAD, _HID), _f32),
      mesh=mesh,
      scratch_types=[
          pltpu.VMEM((_CPT, _CH), jnp.int32),
          pltpu.VMEM((_CPT, _CH), jnp.int32),
          pltpu.VMEM((_NBUF, _CH, _HID), _f32),
      ] + [pltpu.SemaphoreType.DMA] * _NBUF,
  )
  def _edge_pre_kernel(xa_hbm, xb_hbm, src2_hbm, dst2_hbm, pre_hbm, sidx_v,
                       didx_v, rows_v, *sems):
      c = lax.axis_index("c")
      s = lax.axis_index("s")
      cbase = c * (_ECH // 2) + s * _CPT
      ebase = cbase * _CH
      pltpu.sync_copy(src2_hbm.at[pl.ds(cbase, _CPT)], sidx_v)
      pltpu.sync_copy(dst2_hbm.at[pl.ds(cbase, _CPT)], didx_v)

      def xa_copy(g, b):
          return pltpu.make_async_copy(xa_hbm.at[sidx_v.at[g]], rows_v.at[b],
                                       sems[b])

      def xb_copy(g, b):
          return pltpu.make_async_copy(xb_hbm.at[didx_v.at[g]], rows_v.at[b],
                                       sems[b])

      def wr_copy(g, b):
          return pltpu.make_async_copy(
              rows_v.at[b], pre_hbm.at[pl.ds(ebase + g * _CH, _CH)], sems[b])

      def step(i, b):
          g1 = i

          @pl.when(jnp.logical_and(g1 >= _NBUF, g1 < _CPT))
          def _():
              wr_copy(g1 - _NBUF, b).wait()

          @pl.when(g1 < _CPT)
          def _():
              pltpu.async_copy(xa_hbm.at[sidx_v.at[g1]], rows_v.at[b],
                               sems[b])

          g2 = i - 1
          b2 = (b - 1) % _NBUF

          @pl.when(jnp.logical_and(g2 >= 0, g2 < _CPT))
          def _():
              xa_copy(g2, b2).wait()
              pltpu.async_copy(xb_hbm.at[didx_v.at[g2]], rows_v.at[b2],
                               sems[b2], add=True)

          g3 = i - 2
          b3 = (b - 2) % _NBUF

          @pl.when(g3 >= 0)
          def _():
              xb_copy(g3, b3).wait()
              pltpu.async_copy(rows_v.at[b3],
                               pre_hbm.at[pl.ds(ebase + g3 * _CH, _CH)],
                               sems[b3])

      def outer(j, carry):
          for b in range(_NBUF):
              g = j * _NBUF + b

              @pl.when(g < _CPT + 2)
              def _():
                  step(g, b)
          return carry
      lax.fori_loop(0, (_CPT + 2 + _NBUF - 1) // _NBUF, outer, 0)
      for b in range(_NBUF):
          g = _CPT - _NBUF + b
          wr_copy(g, g % _NBUF).wait()

  return _edge_pre_kernel


def _prep0_body(lat_ref, deg_ref, w_ref, y_ref, dinv_ref):
    d = deg_ref[0, :, 0:1] + deg_ref[1, :, 0:1] + 1.0
    dinv = lax.rsqrt(d)
    y_ref[...] = dinv * jnp.dot(lat_ref[...], w_ref[...],
                                preferred_element_type=_f32)
    dinv_ref[...] = dinv


def _prep0(lat, deg, w):
    return pl.pallas_call(
        _prep0_body,
        grid=(_N // _NB,),
        in_specs=[
            pl.BlockSpec((_NB, _LAT), lambda i: (i, 0)),
            pl.BlockSpec((2, _NB, 16), lambda i: (0, i, 0)),
            pl.BlockSpec((_LAT, _HID), lambda i: (0, 0)),
        ],
        out_specs=[
            pl.BlockSpec((_NB, _HID), lambda i: (i, 0)),
            pl.BlockSpec((_NB, 1), lambda i: (i, 0)),
        ],
        out_shape=[
            jax.ShapeDtypeStruct((_N, _HID), _f32),
            jax.ShapeDtypeStruct((_N, 1), _f32),
        ],
    )(lat, deg, w)


def _mid_body(agg_ref, y_ref, dinv_ref, b_ref, w_ref, yn_ref):
    dinv = dinv_ref[...]
    x = jnp.maximum(dinv * (agg_ref[0] + agg_ref[1] + y_ref[...]) + b_ref[...],
                    0.0)
    yn_ref[...] = dinv * jnp.dot(x, w_ref[...], preferred_element_type=_f32)


def _mid(agg, y, dinv, b, w):
    return pl.pallas_call(
        _mid_body,
        grid=(_N // _NB,),
        in_specs=[
            pl.BlockSpec((2, _NB, _HID), lambda i: (0, i, 0)),
            pl.BlockSpec((_NB, _HID), lambda i: (i, 0)),
            pl.BlockSpec((_NB, 1), lambda i: (i, 0)),
            pl.BlockSpec((1, _HID), lambda i: (0, 0)),
            pl.BlockSpec((_HID, _HID), lambda i: (0, 0)),
        ],
        out_specs=pl.BlockSpec((_NB, _HID), lambda i: (i, 0)),
        out_shape=jax.ShapeDtypeStruct((_N, _HID), _f32),
    )(agg, y, dinv, b, w)


def _final_body(agg_ref, y_ref, dinv_ref, bg_ref, w1a_ref, bep1_ref, w1b_ref,
                wc1_ref, bc1_ref, wc2_ref, bc2_ref, wu1_ref, bu1_ref, wu2_ref,
                bu2_ref, x_ref, xa_ref, xb_ref, cls_ref, unc_ref):
    dinv = dinv_ref[...]
    x = jnp.maximum(dinv * (agg_ref[0] + agg_ref[1] + y_ref[...]) + bg_ref[...],
                    0.0)
    x_ref[...] = x
    xa_ref[...] = jnp.dot(x, w1a_ref[...], preferred_element_type=_f32) \
        + bep1_ref[...]
    xb_ref[...] = jnp.dot(x, w1b_ref[...], preferred_element_type=_f32)
    hc = jnp.maximum(jnp.dot(x, wc1_ref[...], preferred_element_type=_f32)
                     + bc1_ref[...], 0.0)
    cls_ref[...] = jnp.dot(hc, wc2_ref[...], preferred_element_type=_f32) \
        + bc2_ref[...]
    hu = jnp.maximum(jnp.dot(x, wu1_ref[...], preferred_element_type=_f32)
                     + bu1_ref[...], 0.0)
    z = jnp.dot(hu, wu2_ref[...], preferred_element_type=_f32) + bu2_ref[...]
    unc_ref[...] = jnp.log(1.0 + jnp.exp(-jnp.abs(z))) + jnp.maximum(z, 0.0)


def _final(agg, y, dinv, bg, w1a, bep1, w1b, wc1, bc1, wc2, bc2, wu1, bu1,
           wu2, bu2):
    full = lambda shape: pl.BlockSpec(shape, lambda i: tuple(0 for _ in shape))
    return pl.pallas_call(
        _final_body,
        grid=(_N // _NB,),
        in_specs=[
            pl.BlockSpec((2, _NB, _HID), lambda i: (0, i, 0)),
            pl.BlockSpec((_NB, _HID), lambda i: (i, 0)),
            pl.BlockSpec((_NB, 1), lambda i: (i, 0)),
            full((1, _HID)),
            full((_HID, _HID)), full((1, _HID)), full((_HID, _HID)),
            full((_HID, _HID // 2)), full((1, _HID // 2)),
            full((_HID // 2, 4)), full((1, 4)),
            full((_HID, _HID // 2)), full((1, _HID // 2)),
            full((_HID // 2, 1)), full((1, 1)),
        ],
        out_specs=[
            pl.BlockSpec((_NB, _HID), lambda i: (i, 0)),
            pl.BlockSpec((_NB, _HID), lambda i: (i, 0)),
            pl.BlockSpec((_NB, _HID), lambda i: (i, 0)),
            pl.BlockSpec((_NB, 4), lambda i: (i, 0)),
            pl.BlockSpec((_NB, 1), lambda i: (i, 0)),
        ],
        out_shape=[
            jax.ShapeDtypeStruct((_N, _HID), _f32),
            jax.ShapeDtypeStruct((_N, _HID), _f32),
            jax.ShapeDtypeStruct((_N, _HID), _f32),
            jax.ShapeDtypeStruct((_N, 4), _f32),
            jax.ShapeDtypeStruct((_N, 1), _f32),
        ],
    )(agg, y, dinv, bg, w1a, bep1, w1b, wc1, bc1, wc2, bc2, wu1, bu1, wu2, bu2)


def _edge_mlp_body(pre_ref, w2_ref, b2_ref, w3_ref, b3_ref, ep_ref):
    p = jnp.maximum(pre_ref[...], 0.0)
    t = jnp.maximum(jnp.dot(p, w2_ref[...], preferred_element_type=_f32)
                    + b2_ref[...], 0.0)
    z = jnp.dot(t, w3_ref[...], preferred_element_type=_f32) + b3_ref[...]
    ep_ref[...] = jax.nn.sigmoid(z)


def _edge_mlp(pre, w2, b2, w3, b3):
    full = lambda shape: pl.BlockSpec(shape, lambda i: tuple(0 for _ in shape))
    return pl.pallas_call(
        _edge_mlp_body,
        grid=(_EPAD // _EB,),
        in_specs=[
            pl.BlockSpec((_EB, _HID), lambda i: (i, 0)),
            full((_HID, _HID // 2)), full((1, _HID // 2)),
            full((_HID // 2, 1)), full((1, 1)),
        ],
        out_specs=pl.BlockSpec((_EB, 1), lambda i: (i, 0)),
        out_shape=jax.ShapeDtypeStruct((_EPAD, 1), _f32),
    )(pre, w2, b2, w3, b3)


def kernel(track_sequences, edge_index, params):
    p = params
    x2 = track_sequences.reshape(_N, _T * _IN)
    src = edge_index[0]
    dst = edge_index[1]
    npad = _EPAD - _E
    src1 = jnp.concatenate([src, jnp.zeros((npad,), jnp.int32)])
    dst1a = jnp.concatenate([dst, jnp.full((npad,), _NPAD - 1, jnp.int32)])
    src2 = src1.reshape(_ECH, _CH)
    dst2a = dst1a.reshape(_ECH, _CH)
    dst2p = jnp.concatenate(
        [dst, jnp.zeros((npad,), jnp.int32)]).reshape(_ECH, _CH)

    lat = _encoder(
        x2,
        p["Wih0"].T, p["Whh0"].T, (p["bih0"] + p["bhh0"])[None, :],
        p["Wih1"].T, p["Whh1"].T, (p["bih1"] + p["bhh1"])[None, :],
        p["We1"].T, p["be1"][None, :], p["We2"].T, p["be2"][None, :],
    )

    deg = _make_deg_kernel()(dst2a)
    y0, dinv = _prep0(lat, deg, p["Wg0"].T)

    agg0 = _make_agg_kernel()(y0, src1, dst1a)
    y1 = _mid(agg0, y0, dinv, p["bg0"][None, :], p["Wg1"].T)
    agg1 = _make_agg_kernel()(y1, src1, dst1a)
    y2 = _mid(agg1, y1, dinv, p["bg1"][None, :], p["Wg2"].T)
    agg2 = _make_agg_kernel()(y2, src1, dst1a)

    x3, xa, xb, cls, unc = _final(
        agg2, y2, dinv, p["bg2"][None, :],
        p["Wep1"][:, :_HID].T, p["bep1"][None, :], p["Wep1"][:, _HID:].T,
        p["Wc1"].T, p["bc1"][None, :], p["Wc2"].T, p["bc2"][None, :],
        p["Wu1"].T, p["bu1"][None, :], p["Wu2"].T, p["bu2"][None, :],
    )

    pre = _make_edge_pre_kernel()(xa, xb, src2, dst2p)
    ep = _edge_mlp(pre, p["Wep2"].T, p["bep2"][None, :],
                   p["Wep3"].T, p["bep3"][None, :])[:_E]

    return (x3, ep, cls, unc)


# speedup vs baseline: 10.3254x; 2.1564x over previous
import functools

import jax
import jax.numpy as jnp
from jax import lax
from jax.experimental import pallas as pl
from jax.experimental.pallas import tpu as pltpu
from jax.experimental.pallas import tpu_sc as plsc

_N = 10000
_T = 20
_IN = 16
_HID = 128
_LAT = 64
_E = 320000

_NB = 400
_CH = 128
_ECH = 2560
_EPAD = _ECH * _CH
_EB = 2048
_CPT = _ECH // 32
_NPAD = 10240
_RPT = _NPAD // 16
_ZCH = 128
_NBUF = 4

_f32 = jnp.float32


def _encoder_body(x_ref, wih0, whh0, b0, wih1, whh1, b1, we1, be1, we2, be2,
                  lat_ref):
    x = x_ref[...]
    nb = x.shape[0]
    h = jnp.zeros((nb, _HID), _f32)
    c = jnp.zeros((nb, _HID), _f32)
    hs = []
    for t in range(_T):
        xt = x[:, t * _IN:(t + 1) * _IN]
        g = (jnp.dot(xt, wih0[...], preferred_element_type=_f32)
             + jnp.dot(h, whh0[...], preferred_element_type=_f32) + b0[...])
        gi = jax.nn.sigmoid(g[:, 0 * _HID:1 * _HID])
        gf = jax.nn.sigmoid(g[:, 1 * _HID:2 * _HID])
        gg = jnp.tanh(g[:, 2 * _HID:3 * _HID])
        go = jax.nn.sigmoid(g[:, 3 * _HID:4 * _HID])
        c = gf * c + gi * gg
        h = go * jnp.tanh(c)
        hs.append(h)
    h = jnp.zeros((nb, _HID), _f32)
    c = jnp.zeros((nb, _HID), _f32)
    for t in range(_T):
        g = (jnp.dot(hs[t], wih1[...], preferred_element_type=_f32)
             + jnp.dot(h, whh1[...], preferred_element_type=_f32) + b1[...])
        gi = jax.nn.sigmoid(g[:, 0 * _HID:1 * _HID])
        gf = jax.nn.sigmoid(g[:, 1 * _HID:2 * _HID])
        gg = jnp.tanh(g[:, 2 * _HID:3 * _HID])
        go = jax.nn.sigmoid(g[:, 3 * _HID:4 * _HID])
        c = gf * c + gi * gg
        h = go * jnp.tanh(c)
    e = jnp.maximum(jnp.dot(h, we1[...], preferred_element_type=_f32)
                    + be1[...], 0.0)
    lat_ref[...] = jnp.dot(e, we2[...], preferred_element_type=_f32) + be2[...]


def _encoder(x2, wih0, whh0, b0, wih1, whh1, b1, we1, be1, we2, be2):
    full = lambda shape: pl.BlockSpec(shape, lambda i: tuple(0 for _ in shape))
    return pl.pallas_call(
        _encoder_body,
        grid=(_N // _NB,),
        in_specs=[
            pl.BlockSpec((_NB, _T * _IN), lambda i: (i, 0)),
            full((_IN, 4 * _HID)), full((_HID, 4 * _HID)), full((1, 4 * _HID)),
            full((_HID, 4 * _HID)), full((_HID, 4 * _HID)), full((1, 4 * _HID)),
            full((_HID, _HID)), full((1, _HID)),
            full((_HID, _LAT)), full((1, _LAT)),
        ],
        out_specs=pl.BlockSpec((_NB, _LAT), lambda i: (i, 0)),
        out_shape=jax.ShapeDtypeStruct((_N, _LAT), _f32),
    )(x2, wih0, whh0, b0, wih1, whh1, b1, we1, be1, we2, be2)


@functools.cache
def _make_deg_kernel():
  mesh = plsc.VectorSubcoreMesh(core_axis_name="c", subcore_axis_name="s")

  @functools.partial(
      pl.kernel,
      out_type=jax.ShapeDtypeStruct((2, _NPAD, 16), _f32),
      mesh=mesh,
      scratch_types=[
          pltpu.VMEM((_CH, 16), _f32),
          pltpu.VMEM((_ZCH, 16), _f32),
          pltpu.VMEM((_CPT, _CH), jnp.int32),
          pltpu.VMEM_SHARED((_NPAD, 16), _f32),
      ],
  )
  def _deg_kernel(dst2_hbm, deg_hbm, ones_v, stage_v, didx_v, acc_sh):
      c = lax.axis_index("c")
      s = lax.axis_index("s")

      def fill(i, carry):
          ones_v[i, :] = jnp.full((16,), 1.0, _f32)
          return carry
      lax.fori_loop(0, _CH, fill, 0)

      def zfill(i, carry):
          stage_v[i, :] = jnp.zeros((16,), _f32)
          return carry
      lax.fori_loop(0, _ZCH, zfill, 0)

      pltpu.sync_copy(dst2_hbm.at[pl.ds(c * (_ECH // 2) + s * _CPT, _CPT)],
                      didx_v)

      row0 = s * _RPT
      for k in range(_RPT // _ZCH):
          pltpu.sync_copy(stage_v, acc_sh.at[pl.ds(row0 + k * _ZCH, _ZCH)])
      plsc.subcore_barrier()

      def chunk(g, carry):
          pltpu.sync_copy(ones_v, acc_sh.at[didx_v.at[g]], add=True)
          return carry
      lax.fori_loop(0, _CPT, chunk, 0)
      plsc.subcore_barrier()

      for k in range(_RPT // _ZCH):
          r0 = row0 + k * _ZCH
          pltpu.sync_copy(acc_sh.at[pl.ds(r0, _ZCH)], stage_v)
          pltpu.sync_copy(stage_v, deg_hbm.at[c, pl.ds(r0, _ZCH)])

  return _deg_kernel


@functools.cache
def _make_agg_kernel():
  mesh = plsc.VectorSubcoreMesh(core_axis_name="c", subcore_axis_name="s")

  @functools.partial(
      pl.kernel,
      out_type=jax.ShapeDtypeStruct((2, _NPAD, _HID), _f32),
      mesh=mesh,
      scratch_types=[
          pltpu.VMEM((4, _CH), jnp.int32),
          pltpu.VMEM((4, _CH), jnp.int32),
          pltpu.VMEM((2, _CH, _HID), _f32),
          pltpu.VMEM_SHARED((_NPAD, _HID), _f32),
      ] + [pltpu.SemaphoreType.DMA] * 6,
  )
  def _agg_kernel(y_hbm, src_hbm, dst_hbm, agg_hbm, sidx_v, didx_v, rows_v,
                  acc_sh, *sems):
      c = lax.axis_index("c")
      s = lax.axis_index("s")
      isem = sems[:4]
      gsem = sems[4:]

      def zfill(i, carry):
          for j in range(_HID // 16):
              rows_v[0, i, pl.ds(j * 16, 16)] = jnp.zeros((16,), _f32)
          return carry
      lax.fori_loop(0, _CH, zfill, 0)

      row0 = s * _RPT
      for k in range(_RPT // _ZCH):
          pltpu.sync_copy(rows_v.at[0], acc_sh.at[pl.ds(row0 + k * _ZCH,
                                                        _ZCH)])
      plsc.subcore_barrier()

      ebase = (c * (_ECH // 2) + s * _CPT) * _CH

      def start_idx(g, b):
          pltpu.async_copy(src_hbm.at[pl.ds(ebase + g * _CH, _CH)],
                           sidx_v.at[b], isem[b])
          pltpu.async_copy(dst_hbm.at[pl.ds(ebase + g * _CH, _CH)],
                           didx_v.at[b], isem[b])

      def wait_idx(g, b):
          pltpu.make_async_copy(src_hbm.at[pl.ds(ebase + g * _CH, _CH)],
                                sidx_v.at[b], isem[b]).wait()
          pltpu.make_async_copy(dst_hbm.at[pl.ds(ebase + g * _CH, _CH)],
                                didx_v.at[b], isem[b]).wait()

      def start_gather(g, bi, br):
          pltpu.async_copy(y_hbm.at[sidx_v.at[bi]], rows_v.at[br], gsem[br])

      def wait_gather(bi, br):
          pltpu.make_async_copy(y_hbm.at[sidx_v.at[bi]], rows_v.at[br],
                                gsem[br]).wait()

      for b in range(4):
          start_idx(b, b)
      for g in range(2):
          wait_idx(g, g)
          start_gather(g, g, g)

      def outer(j, carry):
          for b in range(4):
              g = j * 4 + b
              br = b % 2
              wait_gather(b, br)
              pltpu.sync_copy(rows_v.at[br], acc_sh.at[didx_v.at[b]],
                              add=True)

              @pl.when(g + 4 < _CPT)
              def _():
                  start_idx(g + 4, b)

              @pl.when(g + 2 < _CPT)
              def _():
                  wait_idx(g + 2, (b + 2) % 4)
                  start_gather(g + 2, (b + 2) % 4, br)
          return carry
      lax.fori_loop(0, _CPT // 4, outer, 0)
      plsc.subcore_barrier()

      for k in range(_RPT // _ZCH):
          r0 = row0 + k * _ZCH
          pltpu.sync_copy(acc_sh.at[pl.ds(r0, _ZCH)], rows_v.at[0])
          pltpu.sync_copy(rows_v.at[0], agg_hbm.at[c, pl.ds(r0, _ZCH)])

  return _agg_kernel


@functools.cache
def _make_edge_pre_kernel():
  mesh = plsc.VectorSubcoreMesh(core_axis_name="c", subcore_axis_name="s")

  @functools.partial(
      pl.kernel,
      out_type=jax.ShapeDtypeStruct((_EPAD, _HID), _f32),
      mesh=mesh,
      scratch_types=[
          pltpu.VMEM((_CPT, _CH), jnp.int32),
          pltpu.VMEM((_CPT, _CH), jnp.int32),
          pltpu.VMEM((_NBUF, _CH, _HID), _f32),
      ] + [pltpu.SemaphoreType.DMA] * _NBUF,
  )
  def _edge_pre_kernel(xa_hbm, xb_hbm, src2_hbm, dst2_hbm, pre_hbm, sidx_v,
                       didx_v, rows_v, *sems):
      c = lax.axis_index("c")
      s = lax.axis_index("s")
      cbase = c * (_ECH // 2) + s * _CPT
      ebase = cbase * _CH
      pltpu.sync_copy(src2_hbm.at[pl.ds(cbase, _CPT)], sidx_v)
      pltpu.sync_copy(dst2_hbm.at[pl.ds(cbase, _CPT)], didx_v)

      def xa_copy(g, b):
          return pltpu.make_async_copy(xa_hbm.at[sidx_v.at[g]], rows_v.at[b],
                                       sems[b])

      def xb_copy(g, b):
          return pltpu.make_async_copy(xb_hbm.at[didx_v.at[g]], rows_v.at[b],
                                       sems[b])

      def wr_copy(g, b):
          return pltpu.make_async_copy(
              rows_v.at[b], pre_hbm.at[pl.ds(ebase + g * _CH, _CH)], sems[b])

      def step(i, b):
          g1 = i

          @pl.when(jnp.logical_and(g1 >= _NBUF, g1 < _CPT))
          def _():
              wr_copy(g1 - _NBUF, b).wait()

          @pl.when(g1 < _CPT)
          def _():
              pltpu.async_copy(xa_hbm.at[sidx_v.at[g1]], rows_v.at[b],
                               sems[b])

          g2 = i - 1
          b2 = (b - 1) % _NBUF

          @pl.when(jnp.logical_and(g2 >= 0, g2 < _CPT))
          def _():
              xa_copy(g2, b2).wait()
              pltpu.async_copy(xb_hbm.at[didx_v.at[g2]], rows_v.at[b2],
                               sems[b2], add=True)

          g3 = i - 2
          b3 = (b - 2) % _NBUF

          @pl.when(g3 >= 0)
          def _():
              xb_copy(g3, b3).wait()
              pltpu.async_copy(rows_v.at[b3],
                               pre_hbm.at[pl.ds(ebase + g3 * _CH, _CH)],
                               sems[b3])

      def outer(j, carry):
          for b in range(_NBUF):
              g = j * _NBUF + b

              @pl.when(g < _CPT + 2)
              def _():
                  step(g, b)
          return carry
      lax.fori_loop(0, (_CPT + 2 + _NBUF - 1) // _NBUF, outer, 0)
      for b in range(_NBUF):
          g = _CPT - _NBUF + b
          wr_copy(g, g % _NBUF).wait()

  return _edge_pre_kernel


def _prep0_body(lat_ref, deg_ref, w_ref, y_ref, dinv_ref):
    d = deg_ref[0, :, 0:1] + deg_ref[1, :, 0:1] + 1.0
    dinv = lax.rsqrt(d)
    y_ref[...] = dinv * jnp.dot(lat_ref[...], w_ref[...],
                                preferred_element_type=_f32)
    dinv_ref[...] = dinv


def _prep0(lat, deg, w):
    return pl.pallas_call(
        _prep0_body,
        grid=(_N // _NB,),
        in_specs=[
            pl.BlockSpec((_NB, _LAT), lambda i: (i, 0)),
            pl.BlockSpec((2, _NB, 16), lambda i: (0, i, 0)),
            pl.BlockSpec((_LAT, _HID), lambda i: (0, 0)),
        ],
        out_specs=[
            pl.BlockSpec((_NB, _HID), lambda i: (i, 0)),
            pl.BlockSpec((_NB, 1), lambda i: (i, 0)),
        ],
        out_shape=[
            jax.ShapeDtypeStruct((_N, _HID), _f32),
            jax.ShapeDtypeStruct((_N, 1), _f32),
        ],
    )(lat, deg, w)


def _mid_body(agg_ref, y_ref, dinv_ref, b_ref, w_ref, yn_ref):
    dinv = dinv_ref[...]
    x = jnp.maximum(dinv * (agg_ref[0] + agg_ref[1] + y_ref[...]) + b_ref[...],
                    0.0)
    yn_ref[...] = dinv * jnp.dot(x, w_ref[...], preferred_element_type=_f32)


def _mid(agg, y, dinv, b, w):
    return pl.pallas_call(
        _mid_body,
        grid=(_N // _NB,),
        in_specs=[
            pl.BlockSpec((2, _NB, _HID), lambda i: (0, i, 0)),
            pl.BlockSpec((_NB, _HID), lambda i: (i, 0)),
            pl.BlockSpec((_NB, 1), lambda i: (i, 0)),
            pl.BlockSpec((1, _HID), lambda i: (0, 0)),
            pl.BlockSpec((_HID, _HID), lambda i: (0, 0)),
        ],
        out_specs=pl.BlockSpec((_NB, _HID), lambda i: (i, 0)),
        out_shape=jax.ShapeDtypeStruct((_N, _HID), _f32),
    )(agg, y, dinv, b, w)


def _final_body(agg_ref, y_ref, dinv_ref, bg_ref, w1a_ref, bep1_ref, w1b_ref,
                wc1_ref, bc1_ref, wc2_ref, bc2_ref, wu1_ref, bu1_ref, wu2_ref,
                bu2_ref, x_ref, xa_ref, xb_ref, cls_ref, unc_ref):
    dinv = dinv_ref[...]
    x = jnp.maximum(dinv * (agg_ref[0] + agg_ref[1] + y_ref[...]) + bg_ref[...],
                    0.0)
    x_ref[...] = x
    xa_ref[...] = jnp.dot(x, w1a_ref[...], preferred_element_type=_f32) \
        + bep1_ref[...]
    xb_ref[...] = jnp.dot(x, w1b_ref[...], preferred_element_type=_f32)
    hc = jnp.maximum(jnp.dot(x, wc1_ref[...], preferred_element_type=_f32)
                     + bc1_ref[...], 0.0)
    cls_ref[...] = jnp.dot(hc, wc2_ref[...], preferred_element_type=_f32) \
        + bc2_ref[...]
    hu = jnp.maximum(jnp.dot(x, wu1_ref[...], preferred_element_type=_f32)
                     + bu1_ref[...], 0.0)
    z = jnp.dot(hu, wu2_ref[...], preferred_element_type=_f32) + bu2_ref[...]
    unc_ref[...] = jnp.log(1.0 + jnp.exp(-jnp.abs(z))) + jnp.maximum(z, 0.0)


def _final(agg, y, dinv, bg, w1a, bep1, w1b, wc1, bc1, wc2, bc2, wu1, bu1,
           wu2, bu2):
    full = lambda shape: pl.BlockSpec(shape, lambda i: tuple(0 for _ in shape))
    return pl.pallas_call(
        _final_body,
        grid=(_N // _NB,),
        in_specs=[
            pl.BlockSpec((2, _NB, _HID), lambda i: (0, i, 0)),
            pl.BlockSpec((_NB, _HID), lambda i: (i, 0)),
            pl.BlockSpec((_NB, 1), lambda i: (i, 0)),
            full((1, _HID)),
            full((_HID, _HID)), full((1, _HID)), full((_HID, _HID)),
            full((_HID, _HID // 2)), full((1, _HID // 2)),
            full((_HID // 2, 4)), full((1, 4)),
            full((_HID, _HID // 2)), full((1, _HID // 2)),
            full((_HID // 2, 1)), full((1, 1)),
        ],
        out_specs=[
            pl.BlockSpec((_NB, _HID), lambda i: (i, 0)),
            pl.BlockSpec((_NB, _HID), lambda i: (i, 0)),
            pl.BlockSpec((_NB, _HID), lambda i: (i, 0)),
            pl.BlockSpec((_NB, 4), lambda i: (i, 0)),
            pl.BlockSpec((_NB, 1), lambda i: (i, 0)),
        ],
        out_shape=[
            jax.ShapeDtypeStruct((_N, _HID), _f32),
            jax.ShapeDtypeStruct((_N, _HID), _f32),
            jax.ShapeDtypeStruct((_N, _HID), _f32),
            jax.ShapeDtypeStruct((_N, 4), _f32),
            jax.ShapeDtypeStruct((_N, 1), _f32),
        ],
    )(agg, y, dinv, bg, w1a, bep1, w1b, wc1, bc1, wc2, bc2, wu1, bu1, wu2, bu2)


def _edge_mlp_body(pre_ref, w2_ref, b2_ref, w3_ref, b3_ref, ep_ref):
    p = jnp.maximum(pre_ref[...], 0.0)
    t = jnp.maximum(jnp.dot(p, w2_ref[...], preferred_element_type=_f32)
                    + b2_ref[...], 0.0)
    z = jnp.dot(t, w3_ref[...], preferred_element_type=_f32) + b3_ref[...]
    ep_ref[...] = jax.nn.sigmoid(z)


def _edge_mlp(pre, w2, b2, w3, b3):
    full = lambda shape: pl.BlockSpec(shape, lambda i: tuple(0 for _ in shape))
    return pl.pallas_call(
        _edge_mlp_body,
        grid=(_EPAD // _EB,),
        in_specs=[
            pl.BlockSpec((_EB, _HID), lambda i: (i, 0)),
            full((_HID, _HID // 2)), full((1, _HID // 2)),
            full((_HID // 2, 1)), full((1, 1)),
        ],
        out_specs=pl.BlockSpec((_EB, 1), lambda i: (i, 0)),
        out_shape=jax.ShapeDtypeStruct((_EPAD, 1), _f32),
    )(pre, w2, b2, w3, b3)


def kernel(track_sequences, edge_index, params):
    p = params
    x2 = track_sequences.reshape(_N, _T * _IN)
    src = edge_index[0]
    dst = edge_index[1]
    npad = _EPAD - _E
    pad_iota = lax.iota(jnp.int32, npad)
    src1 = jnp.concatenate([src, pad_iota % _N])
    dst1a = jnp.concatenate([dst, _N + pad_iota % (_NPAD - _N)])
    src2 = src1.reshape(_ECH, _CH)
    dst2a = dst1a.reshape(_ECH, _CH)
    dst2p = jnp.concatenate([dst, pad_iota % _N]).reshape(_ECH, _CH)

    lat = _encoder(
        x2,
        p["Wih0"].T, p["Whh0"].T, (p["bih0"] + p["bhh0"])[None, :],
        p["Wih1"].T, p["Whh1"].T, (p["bih1"] + p["bhh1"])[None, :],
        p["We1"].T, p["be1"][None, :], p["We2"].T, p["be2"][None, :],
    )

    deg = _make_deg_kernel()(dst2a)
    y0, dinv = _prep0(lat, deg, p["Wg0"].T)

    agg0 = _make_agg_kernel()(y0, src1, dst1a)
    y1 = _mid(agg0, y0, dinv, p["bg0"][None, :], p["Wg1"].T)
    agg1 = _make_agg_kernel()(y1, src1, dst1a)
    y2 = _mid(agg1, y1, dinv, p["bg1"][None, :], p["Wg2"].T)
    agg2 = _make_agg_kernel()(y2, src1, dst1a)

    x3, xa, xb, cls, unc = _final(
        agg2, y2, dinv, p["bg2"][None, :],
        p["Wep1"][:, :_HID].T, p["bep1"][None, :], p["Wep1"][:, _HID:].T,
        p["Wc1"].T, p["bc1"][None, :], p["Wc2"].T, p["bc2"][None, :],
        p["Wu1"].T, p["bu1"][None, :], p["Wu2"].T, p["bu2"][None, :],
    )

    pre = _make_edge_pre_kernel()(xa, xb, src2, dst2p)
    ep = _edge_mlp(pre, p["Wep2"].T, p["bep2"][None, :],
                   p["Wep3"].T, p["bep3"][None, :])[:_E]

    return (x3, ep, cls, unc)
